# baseline (speedup 1.0000x reference)
import functools

import jax
import jax.numpy as jnp
from jax import lax
from jax.experimental import pallas as pl
from jax.experimental.pallas import tpu as pltpu
from jax.experimental.pallas import tpu_sc as plsc

_B, _C, _H, _W = 8, 19, 512, 512
_NW = 32
_NBLK = _B * (_H // 8) * (_W // 128)
_NBLK_SC = 1536
_BPW = _NBLK_SC // _NW
_SQRTHF_BITS = 0x3F3504F3


def _log_parts(v):
    bits = lax.bitcast_convert_type(v, jnp.int32)
    bias = bits - _SQRTHF_BITS
    e = lax.shift_right_arithmetic(bias, 23)
    m = lax.bitcast_convert_type(
        jnp.bitwise_and(bias, 0x007FFFFF) + _SQRTHF_BITS, jnp.float32)
    f = m - jnp.float32(1.0)
    z = f * f
    y = f * jnp.float32(-1.6668057665e-1) + jnp.float32(2.0000714765e-1)
    y = y * f + jnp.float32(-2.4999993993e-1)
    y = y * f + jnp.float32(3.3333331174e-1)
    y = y * f * z - jnp.float32(0.5) * z
    return f + y, e


_mesh = plsc.VectorSubcoreMesh(core_axis_name="c", subcore_axis_name="s")


@functools.partial(
    pl.kernel,
    out_type=(jax.ShapeDtypeStruct((_NW, 16), jnp.float32),
              jax.ShapeDtypeStruct((_NW, 16), jnp.int32),
              jax.ShapeDtypeStruct((_NW, 16), jnp.float32)),
    mesh=_mesh,
    scratch_types=[
        pltpu.VMEM((4 * _C * 8, 128), jnp.float32),
        pltpu.VMEM((4, 8, 128), jnp.int32),
        pltpu.VMEM((16,), jnp.float32),
        pltpu.VMEM((16,), jnp.int32),
        pltpu.VMEM((16,), jnp.float32),
        pltpu.SemaphoreType.DMA,
        pltpu.SemaphoreType.DMA,
        pltpu.SemaphoreType.DMA,
        pltpu.SemaphoreType.DMA,
    ],
    compiler_params=pltpu.CompilerParams(needs_layout_passes=False),
)
def _nll_sc(score_4d, tgt_3d, fsum_out, esum_out, cnt_out,
            cls_v, tgt_v, fs_v, es_v, cn_v, *sems):
    wid = lax.axis_index("s") * 2 + lax.axis_index("c")
    lanes = lax.broadcasted_iota(jnp.int32, (16,), 0)

    def _fire(bi, slot):
        g = wid * _BPW + bi
        b = lax.shift_right_logical(g, 8)
        h0 = lax.bitwise_and(lax.shift_right_logical(g, 2), 63) * 8
        w0 = lax.bitwise_and(g, 3) * 128
        for c in range(_C):
            pltpu.make_async_copy(
                score_4d.at[b, c, pl.ds(h0, 8), pl.ds(w0, 128)],
                cls_v.at[pl.ds((slot * _C + c) * 8, 8)], sems[slot]).start()
        pltpu.make_async_copy(
            tgt_3d.at[b, pl.ds(h0, 8), pl.ds(w0, 128)], tgt_v.at[slot],
            sems[slot]).start()

    def _drain(slot):
        pltpu.make_async_copy(
            score_4d.at[0, 0, pl.ds(0, _C * 8), pl.ds(0, 128)],
            cls_v.at[pl.ds(slot * _C * 8, _C * 8)], sems[slot]).wait()
        pltpu.make_async_copy(
            tgt_3d.at[0, pl.ds(0, 8), pl.ds(0, 128)], tgt_v.at[slot],
            sems[slot]).wait()

    def _reduce(slot, carry):
        row_base = slot * (_C * 8)

        @plsc.parallel_loop(0, 64, carry=carry, unroll=8)
        def red_body(g, c):
            afy, ae, n = c
            hl = lax.shift_right_logical(g, 3)
            wj = lax.bitwise_and(g, 7)
            t = tgt_v[slot, hl, pl.ds(wj * 16, 16)]
            v = plsc.load_gather(
                cls_v, [(row_base + hl) + t * 8, wj * 16 + lanes])
            fy, e = _log_parts(v)
            return (afy + fy, ae + e,
                    n + jnp.where(v != jnp.float32(1.0),
                                  jnp.float32(1.0), jnp.float32(0.0)))

        return red_body

    def outer(it, carry):
        for b in range(4):
            bi = it * 4 + b
            nxt = bi + 3

            @pl.when(nxt < _BPW)
            def _():
                _fire(nxt, (b + 3) % 4)

            _drain(b)
            carry = _reduce(b, carry)
        return carry

    for s in range(3):
        _fire(s, s)
    zf = jnp.zeros((16,), jnp.float32)
    zi = jnp.zeros((16,), jnp.int32)
    afy, ae, cnt = lax.fori_loop(0, _BPW // 4, outer, (zf, zi, zf))
    fs_v[...] = afy
    es_v[...] = ae
    cn_v[...] = cnt
    pltpu.sync_copy(fs_v, fsum_out.at[wid])
    pltpu.sync_copy(es_v, esum_out.at[wid])
    pltpu.sync_copy(cn_v, cnt_out.at[wid])


_TCH = 128
_GPS = _TCH // 8 * 4
_TC_GRID = (_NBLK - _NBLK_SC) // _GPS


def _tc_body(score_ref, tgt_ref, sum_ref, cnt_ref):
    i = pl.program_id(0)
    t = tgt_ref[0]
    picked = jnp.zeros((_TCH, _W), jnp.float32)
    for c in range(_C):
        picked = picked + jnp.where(t == c, score_ref[0, c], jnp.float32(0.0))
    lg = jnp.log(picked)

    @pl.when(i == 0)
    def _():
        sum_ref[...] = jnp.zeros((_TCH, _W), jnp.float32)
        cnt_ref[...] = jnp.zeros((_TCH, _W), jnp.float32)

    sum_ref[...] += lg
    cnt_ref[...] += jnp.where(picked != jnp.float32(1.0),
                              jnp.float32(1.0), jnp.float32(0.0))


_HB = _H // _TCH


def _tc_index4(i):
    g0 = _NBLK_SC + i * _GPS
    return (g0 // 256, 0, (g0 % 256) // _GPS, 0)


def _tc_index3(i):
    idx = _tc_index4(i)
    return (idx[0], idx[2], idx[3])


_nll_tc = pl.pallas_call(
    _tc_body,
    grid=(_TC_GRID,),
    in_specs=[
        pl.BlockSpec((1, _C, _TCH, _W), _tc_index4),
        pl.BlockSpec((1, _TCH, _W), _tc_index3),
    ],
    out_specs=[
        pl.BlockSpec((_TCH, _W), lambda i: (0, 0)),
        pl.BlockSpec((_TCH, _W), lambda i: (0, 0)),
    ],
    out_shape=[jax.ShapeDtypeStruct((_TCH, _W), jnp.float32),
               jax.ShapeDtypeStruct((_TCH, _W), jnp.float32)],
    compiler_params=pltpu.CompilerParams(
        dimension_semantics=("arbitrary",)),
)


def kernel(score, target):
    fsum, esum, cnts = _nll_sc(score, target)
    tc_sum, tc_cnt = _nll_tc(score, target)
    total = (jnp.sum(fsum)
             + jnp.float32(0.6931471805599453)
             * jnp.sum(esum).astype(jnp.float32)
             + jnp.sum(tc_sum))
    return -total / (jnp.sum(cnts) + jnp.sum(tc_cnt))


# speedup vs baseline: 1.0347x; 1.0347x over previous
import functools

import jax
import jax.numpy as jnp
from jax import lax
from jax.experimental import pallas as pl
from jax.experimental.pallas import tpu as pltpu
from jax.experimental.pallas import tpu_sc as plsc

_B, _C, _H, _W = 8, 19, 512, 512
_NW = 32
_NBLK = _B * (_H // 8) * (_W // 128)
_NBLK_SC = 1152
_BPW = _NBLK_SC // _NW
_SQRTHF_BITS = 0x3F3504F3


def _log_parts(v):
    bits = lax.bitcast_convert_type(v, jnp.int32)
    bias = bits - _SQRTHF_BITS
    e = lax.shift_right_arithmetic(bias, 23)
    m = lax.bitcast_convert_type(
        jnp.bitwise_and(bias, 0x007FFFFF) + _SQRTHF_BITS, jnp.float32)
    f = m - jnp.float32(1.0)
    z = f * f
    y = f * jnp.float32(-1.6668057665e-1) + jnp.float32(2.0000714765e-1)
    y = y * f + jnp.float32(-2.4999993993e-1)
    y = y * f + jnp.float32(3.3333331174e-1)
    y = y * f * z - jnp.float32(0.5) * z
    return f + y, e


_mesh = plsc.VectorSubcoreMesh(core_axis_name="c", subcore_axis_name="s")


@functools.partial(
    pl.kernel,
    out_type=(jax.ShapeDtypeStruct((_NW, 16), jnp.float32),
              jax.ShapeDtypeStruct((_NW, 16), jnp.int32),
              jax.ShapeDtypeStruct((_NW, 16), jnp.float32)),
    mesh=_mesh,
    scratch_types=[
        pltpu.VMEM((4 * _C * 8, 128), jnp.float32),
        pltpu.VMEM((4, 8, 128), jnp.int32),
        pltpu.VMEM((16,), jnp.float32),
        pltpu.VMEM((16,), jnp.int32),
        pltpu.VMEM((16,), jnp.float32),
        pltpu.SemaphoreType.DMA,
        pltpu.SemaphoreType.DMA,
        pltpu.SemaphoreType.DMA,
        pltpu.SemaphoreType.DMA,
    ],
    compiler_params=pltpu.CompilerParams(needs_layout_passes=False),
)
def _nll_sc(score_4d, tgt_3d, fsum_out, esum_out, cnt_out,
            cls_v, tgt_v, fs_v, es_v, cn_v, *sems):
    wid = lax.axis_index("s") * 2 + lax.axis_index("c")
    lanes = lax.broadcasted_iota(jnp.int32, (16,), 0)

    def _fire(bi, slot):
        g = wid * _BPW + bi
        b = lax.shift_right_logical(g, 8)
        h0 = lax.bitwise_and(lax.shift_right_logical(g, 2), 63) * 8
        w0 = lax.bitwise_and(g, 3) * 128
        for c in range(_C):
            pltpu.make_async_copy(
                score_4d.at[b, c, pl.ds(h0, 8), pl.ds(w0, 128)],
                cls_v.at[pl.ds((slot * _C + c) * 8, 8)], sems[slot]).start()
        pltpu.make_async_copy(
            tgt_3d.at[b, pl.ds(h0, 8), pl.ds(w0, 128)], tgt_v.at[slot],
            sems[slot]).start()

    def _drain(slot):
        pltpu.make_async_copy(
            score_4d.at[0, 0, pl.ds(0, _C * 8), pl.ds(0, 128)],
            cls_v.at[pl.ds(slot * _C * 8, _C * 8)], sems[slot]).wait()
        pltpu.make_async_copy(
            tgt_3d.at[0, pl.ds(0, 8), pl.ds(0, 128)], tgt_v.at[slot],
            sems[slot]).wait()

    def _reduce(slot, carry):
        row_base = slot * (_C * 8)

        @plsc.parallel_loop(0, 64, carry=carry, unroll=8)
        def red_body(g, c):
            afy, ae, n = c
            hl = lax.shift_right_logical(g, 3)
            wj = lax.bitwise_and(g, 7)
            t = tgt_v[slot, hl, pl.ds(wj * 16, 16)]
            v = plsc.load_gather(
                cls_v, [(row_base + hl) + t * 8, wj * 16 + lanes])
            fy, e = _log_parts(v)
            return (afy + fy, ae + e,
                    n + jnp.where(v != jnp.float32(1.0),
                                  jnp.float32(1.0), jnp.float32(0.0)))

        return red_body

    def outer(it, carry):
        for b in range(4):
            bi = it * 4 + b
            nxt = bi + 3

            @pl.when(nxt < _BPW)
            def _():
                _fire(nxt, (b + 3) % 4)

            _drain(b)
            carry = _reduce(b, carry)
        return carry

    for s in range(3):
        _fire(s, s)
    zf = jnp.zeros((16,), jnp.float32)
    zi = jnp.zeros((16,), jnp.int32)
    afy, ae, cnt = lax.fori_loop(0, _BPW // 4, outer, (zf, zi, zf))
    fs_v[...] = afy
    es_v[...] = ae
    cn_v[...] = cnt
    pltpu.sync_copy(fs_v, fsum_out.at[wid])
    pltpu.sync_copy(es_v, esum_out.at[wid])
    pltpu.sync_copy(cn_v, cnt_out.at[wid])


_TCH = 128
_GPS = _TCH // 8 * 4
_TC_GRID = (_NBLK - _NBLK_SC) // _GPS


def _tc_body(score_ref, tgt_ref, sum_ref, cnt_ref):
    i = pl.program_id(0)
    t = tgt_ref[0]
    picked = jnp.zeros((_TCH, _W), jnp.float32)
    for c in range(_C):
        picked = picked + jnp.where(t == c, score_ref[0, c], jnp.float32(0.0))
    lg = jnp.log(picked)

    @pl.when(i == 0)
    def _():
        sum_ref[...] = jnp.zeros((_TCH, _W), jnp.float32)
        cnt_ref[...] = jnp.zeros((_TCH, _W), jnp.float32)

    sum_ref[...] += lg
    cnt_ref[...] += jnp.where(picked != jnp.float32(1.0),
                              jnp.float32(1.0), jnp.float32(0.0))


_HB = _H // _TCH


def _tc_index4(i):
    g0 = _NBLK_SC + i * _GPS
    return (g0 // 256, 0, (g0 % 256) // _GPS, 0)


def _tc_index3(i):
    idx = _tc_index4(i)
    return (idx[0], idx[2], idx[3])


_nll_tc = pl.pallas_call(
    _tc_body,
    grid=(_TC_GRID,),
    in_specs=[
        pl.BlockSpec((1, _C, _TCH, _W), _tc_index4),
        pl.BlockSpec((1, _TCH, _W), _tc_index3),
    ],
    out_specs=[
        pl.BlockSpec((_TCH, _W), lambda i: (0, 0)),
        pl.BlockSpec((_TCH, _W), lambda i: (0, 0)),
    ],
    out_shape=[jax.ShapeDtypeStruct((_TCH, _W), jnp.float32),
               jax.ShapeDtypeStruct((_TCH, _W), jnp.float32)],
    compiler_params=pltpu.CompilerParams(
        dimension_semantics=("arbitrary",)),
)


def kernel(score, target):
    fsum, esum, cnts = _nll_sc(score, target)
    tc_sum, tc_cnt = _nll_tc(score, target)
    total = (jnp.sum(fsum)
             + jnp.float32(0.6931471805599453)
             * jnp.sum(esum).astype(jnp.float32)
             + jnp.sum(tc_sum))
    return -total / (jnp.sum(cnts) + jnp.sum(tc_cnt))


# speedup vs baseline: 1.0348x; 1.0000x over previous
import functools

import jax
import jax.numpy as jnp
from jax import lax
from jax.experimental import pallas as pl
from jax.experimental.pallas import tpu as pltpu
from jax.experimental.pallas import tpu_sc as plsc

_B, _C, _H, _W = 8, 19, 512, 512
_NW = 32
_NBLK = _B * (_H // 8) * (_W // 128)
_NBLK_SC = 1280
_BPW = _NBLK_SC // _NW
_SQRTHF_BITS = 0x3F3504F3


def _log_parts(v):
    bits = lax.bitcast_convert_type(v, jnp.int32)
    bias = bits - _SQRTHF_BITS
    e = lax.shift_right_arithmetic(bias, 23)
    m = lax.bitcast_convert_type(
        jnp.bitwise_and(bias, 0x007FFFFF) + _SQRTHF_BITS, jnp.float32)
    f = m - jnp.float32(1.0)
    z = f * f
    y = f * jnp.float32(-1.6668057665e-1) + jnp.float32(2.0000714765e-1)
    y = y * f + jnp.float32(-2.4999993993e-1)
    y = y * f + jnp.float32(3.3333331174e-1)
    y = y * f * z - jnp.float32(0.5) * z
    return f + y, e


_mesh = plsc.VectorSubcoreMesh(core_axis_name="c", subcore_axis_name="s")


@functools.partial(
    pl.kernel,
    out_type=(jax.ShapeDtypeStruct((_NW, 16), jnp.float32),
              jax.ShapeDtypeStruct((_NW, 16), jnp.int32),
              jax.ShapeDtypeStruct((_NW, 16), jnp.float32)),
    mesh=_mesh,
    scratch_types=[
        pltpu.VMEM((4 * _C * 8, 128), jnp.float32),
        pltpu.VMEM((4, 8, 128), jnp.int32),
        pltpu.VMEM((16,), jnp.float32),
        pltpu.VMEM((16,), jnp.int32),
        pltpu.VMEM((16,), jnp.float32),
        pltpu.SemaphoreType.DMA,
        pltpu.SemaphoreType.DMA,
        pltpu.SemaphoreType.DMA,
        pltpu.SemaphoreType.DMA,
    ],
    compiler_params=pltpu.CompilerParams(needs_layout_passes=False),
)
def _nll_sc(score_4d, tgt_3d, fsum_out, esum_out, cnt_out,
            cls_v, tgt_v, fs_v, es_v, cn_v, *sems):
    wid = lax.axis_index("s") * 2 + lax.axis_index("c")
    lanes = lax.broadcasted_iota(jnp.int32, (16,), 0)

    def _fire(bi, slot):
        g = wid * _BPW + bi
        b = lax.shift_right_logical(g, 8)
        h0 = lax.bitwise_and(lax.shift_right_logical(g, 2), 63) * 8
        w0 = lax.bitwise_and(g, 3) * 128
        for c in range(_C):
            pltpu.make_async_copy(
                score_4d.at[b, c, pl.ds(h0, 8), pl.ds(w0, 128)],
                cls_v.at[pl.ds((slot * _C + c) * 8, 8)], sems[slot]).start()
        pltpu.make_async_copy(
            tgt_3d.at[b, pl.ds(h0, 8), pl.ds(w0, 128)], tgt_v.at[slot],
            sems[slot]).start()

    def _drain(slot):
        pltpu.make_async_copy(
            score_4d.at[0, 0, pl.ds(0, _C * 8), pl.ds(0, 128)],
            cls_v.at[pl.ds(slot * _C * 8, _C * 8)], sems[slot]).wait()
        pltpu.make_async_copy(
            tgt_3d.at[0, pl.ds(0, 8), pl.ds(0, 128)], tgt_v.at[slot],
            sems[slot]).wait()

    def _reduce(slot, carry):
        row_base = slot * (_C * 8)

        @plsc.parallel_loop(0, 64, carry=carry, unroll=8)
        def red_body(g, c):
            afy, ae, n = c
            hl = lax.shift_right_logical(g, 3)
            wj = lax.bitwise_and(g, 7)
            t = tgt_v[slot, hl, pl.ds(wj * 16, 16)]
            v = plsc.load_gather(
                cls_v, [(row_base + hl) + t * 8, wj * 16 + lanes])
            fy, e = _log_parts(v)
            return (afy + fy, ae + e,
                    n + jnp.where(v != jnp.float32(1.0),
                                  jnp.float32(1.0), jnp.float32(0.0)))

        return red_body

    def outer(it, carry):
        for b in range(4):
            bi = it * 4 + b
            nxt = bi + 3

            @pl.when(nxt < _BPW)
            def _():
                _fire(nxt, (b + 3) % 4)

            _drain(b)
            carry = _reduce(b, carry)
        return carry

    for s in range(3):
        _fire(s, s)
    zf = jnp.zeros((16,), jnp.float32)
    zi = jnp.zeros((16,), jnp.int32)
    afy, ae, cnt = lax.fori_loop(0, _BPW // 4, outer, (zf, zi, zf))
    fs_v[...] = afy
    es_v[...] = ae
    cn_v[...] = cnt
    pltpu.sync_copy(fs_v, fsum_out.at[wid])
    pltpu.sync_copy(es_v, esum_out.at[wid])
    pltpu.sync_copy(cn_v, cnt_out.at[wid])


_TCH = 256
_GPS = _TCH // 8 * 4
_TC_GRID = (_NBLK - _NBLK_SC) // _GPS


def _tc_body(score_ref, tgt_ref, sum_ref, cnt_ref):
    i = pl.program_id(0)
    t = tgt_ref[0]
    picked = jnp.zeros((_TCH, _W), jnp.float32)
    for c in range(_C):
        picked = picked + jnp.where(t == c, score_ref[0, c], jnp.float32(0.0))
    lg = jnp.log(picked)

    @pl.when(i == 0)
    def _():
        sum_ref[...] = jnp.zeros((_TCH, _W), jnp.float32)
        cnt_ref[...] = jnp.zeros((_TCH, _W), jnp.float32)

    sum_ref[...] += lg
    cnt_ref[...] += jnp.where(picked != jnp.float32(1.0),
                              jnp.float32(1.0), jnp.float32(0.0))


_HB = _H // _TCH


def _tc_index4(i):
    g0 = _NBLK_SC + i * _GPS
    return (g0 // 256, 0, (g0 % 256) // _GPS, 0)


def _tc_index3(i):
    idx = _tc_index4(i)
    return (idx[0], idx[2], idx[3])


_nll_tc = pl.pallas_call(
    _tc_body,
    grid=(_TC_GRID,),
    in_specs=[
        pl.BlockSpec((1, _C, _TCH, _W), _tc_index4),
        pl.BlockSpec((1, _TCH, _W), _tc_index3),
    ],
    out_specs=[
        pl.BlockSpec((_TCH, _W), lambda i: (0, 0)),
        pl.BlockSpec((_TCH, _W), lambda i: (0, 0)),
    ],
    out_shape=[jax.ShapeDtypeStruct((_TCH, _W), jnp.float32),
               jax.ShapeDtypeStruct((_TCH, _W), jnp.float32)],
    compiler_params=pltpu.CompilerParams(
        dimension_semantics=("arbitrary",)),
)


def kernel(score, target):
    fsum, esum, cnts = _nll_sc(score, target)
    tc_sum, tc_cnt = _nll_tc(score, target)
    total = (jnp.sum(fsum)
             + jnp.float32(0.6931471805599453)
             * jnp.sum(esum).astype(jnp.float32)
             + jnp.sum(tc_sum))
    return -total / (jnp.sum(cnts) + jnp.sum(tc_cnt))
